# Initial kernel scaffold; baseline (speedup 1.0000x reference)
import jax
import jax.numpy as jnp
from jax.experimental import pallas as pl


def kernel(x_n1, x_n2, params, edge_index_n1_n2, edge_index_n2_n1, pair_index):
    raise NotImplementedError("write your pallas kernel here")


# speedup vs baseline: 34.6882x; 34.6882x over previous
import functools

import jax
import jax.numpy as jnp
from jax import lax
from jax.experimental import pallas as pl
from jax.experimental.pallas import tpu as pltpu
from jax.experimental.pallas import tpu_sc as plsc

N = 10000
DF = 128
HID = 128
H = 8
D = 16
L = 2
NPAD = 10240
NC, NS = 2, 16
EB = 2048
NB = 1000


def _mm_body(x_ref, w_ref, b_ref, o_ref, *, act):
    y = jnp.dot(x_ref[...], w_ref[...], preferred_element_type=jnp.float32)
    y = y + b_ref[...]
    if act == "relu":
        y = jnp.maximum(y, 0.0)
    o_ref[...] = y


def _matmul(x, w, b, act=None, block=NB):
    n, fin = x.shape
    fout = w.shape[1]
    return pl.pallas_call(
        functools.partial(_mm_body, act=act),
        grid=(n // block,),
        in_specs=[pl.BlockSpec((block, fin), lambda i: (i, 0)),
                  pl.BlockSpec((fin, fout), lambda i: (0, 0)),
                  pl.BlockSpec((1, fout), lambda i: (0, 0))],
        out_specs=pl.BlockSpec((block, fout), lambda i: (i, 0)),
        out_shape=jax.ShapeDtypeStruct((n, fout), jnp.float32),
    )(x, w, b.reshape(1, fout))


def _head_sum_mat():
    r = lax.broadcasted_iota(jnp.int32, (HID, 16), 0) // D
    c = lax.broadcasted_iota(jnp.int32, (HID, 16), 1)
    return (r == c).astype(jnp.float32)


def _head_bcast_mat():
    r = lax.broadcasted_iota(jnp.int32, (16, HID), 0)
    c = lax.broadcasted_iota(jnp.int32, (16, HID), 1) // D
    return (r == c).astype(jnp.float32)


def _alpha_body(kg_ref, qg_ref, prel_ref, a_ref, m_ref):
    i = pl.program_id(0)
    prod = kg_ref[...] * qg_ref[...]
    a16 = jnp.dot(prod, _head_sum_mat(), preferred_element_type=jnp.float32)
    a16 = a16 * prel_ref[...]
    a_ref[...] = a16
    bm = jnp.max(a16).reshape(1, 1)

    @pl.when(i == 0)
    def _():
        m_ref[...] = bm

    @pl.when(i > 0)
    def _():
        m_ref[...] = jnp.maximum(m_ref[...], bm)


def _alpha(kvg, qg, prel16):
    e = qg.shape[0]
    return pl.pallas_call(
        _alpha_body,
        grid=(e // EB,),
        in_specs=[pl.BlockSpec((EB, HID), lambda i: (i, 0)),
                  pl.BlockSpec((EB, HID), lambda i: (i, 0)),
                  pl.BlockSpec((1, 16), lambda i: (0, 0))],
        out_specs=[pl.BlockSpec((EB, 16), lambda i: (i, 0)),
                   pl.BlockSpec((1, 1), lambda i: (0, 0))],
        out_shape=[jax.ShapeDtypeStruct((e, 16), jnp.float32),
                   jax.ShapeDtypeStruct((1, 1), jnp.float32)],
    )(kvg, qg, prel16)


def _wv_body(a_ref, vg_ref, m_ref, o_ref):
    ex16 = jnp.exp(a_ref[...] - m_ref[...])
    exfull = jnp.dot(ex16, _head_bcast_mat(), preferred_element_type=jnp.float32)
    o_ref[...] = jnp.stack([vg_ref[...] * exfull, exfull])


def _weighted_v(alpha16, kvg, gmax):
    e = alpha16.shape[0]
    return pl.pallas_call(
        _wv_body,
        grid=(e // EB,),
        in_specs=[pl.BlockSpec((EB, 16), lambda i: (i, 0)),
                  pl.BlockSpec((EB, HID), lambda i: (i, 1)),
                  pl.BlockSpec((1, 1), lambda i: (0, 0))],
        out_specs=pl.BlockSpec((2, EB, HID), lambda i: (0, i, 0)),
        out_shape=jax.ShapeDtypeStruct((2, e, HID), jnp.float32),
    )(alpha16, kvg, gmax)


def _fin_body(nd_ref, xd_ref, wa_ref, ba_ref, beta_ref, o_ref):
    parts = nd_ref[...]
    num = parts[0]
    denfull = parts[1]
    agg = num / (denfull + 1e-16)
    g = jax.nn.gelu(agg)
    a = jnp.dot(g, wa_ref[...], preferred_element_type=jnp.float32) + ba_ref[...]
    beta = beta_ref[...]
    o_ref[...] = beta * a + (1.0 - beta) * xd_ref[...]


def _finalize(numden, xd, wa, ba, beta):
    return pl.pallas_call(
        _fin_body,
        grid=(N // NB,),
        in_specs=[pl.BlockSpec((2, NB, HID), lambda i: (0, i, 0)),
                  pl.BlockSpec((NB, HID), lambda i: (i, 0)),
                  pl.BlockSpec((HID, HID), lambda i: (0, 0)),
                  pl.BlockSpec((1, HID), lambda i: (0, 0)),
                  pl.BlockSpec((1, 1), lambda i: (0, 0))],
        out_specs=pl.BlockSpec((NB, HID), lambda i: (i, 0)),
        out_shape=jax.ShapeDtypeStruct((N, HID), jnp.float32),
    )(numden, xd, wa, ba.reshape(1, HID), beta.reshape(1, 1))


def _dot_body(g1_ref, g2_ref, o_ref):
    o_ref[...] = jnp.sum(g1_ref[...] * g2_ref[...], axis=1, keepdims=True)


def _pair_dot(g1, g2, npairs, block=NB):
    width = g1.shape[1]
    return pl.pallas_call(
        _dot_body,
        grid=(npairs // block,),
        in_specs=[pl.BlockSpec((block, width), lambda i: (i, 0)),
                  pl.BlockSpec((block, width), lambda i: (i, 0))],
        out_specs=pl.BlockSpec((block, 1), lambda i: (i, 0)),
        out_shape=jax.ShapeDtypeStruct((npairs, 1), jnp.float32),
    )(g1, g2)


def _sc_gather(table, idx2d):
    n, d = table.shape
    e = idx2d.shape[1]
    win = 128
    mesh = plsc.VectorSubcoreMesh(core_axis_name="c", subcore_axis_name="s")

    @functools.partial(pl.kernel, mesh=mesh,
                       out_type=jax.ShapeDtypeStruct((e, d), table.dtype))
    def k(tab_hbm, i_hbm, o_hbm):
        def body(i_vmem, o_vmem):
            pltpu.sync_copy(tab_hbm.at[i_vmem.at[0]], o_vmem)

        pltpu.emit_pipeline(
            body,
            grid=(e // win,),
            in_specs=[pl.BlockSpec((1, win), lambda i: (0, i))],
            out_specs=[pl.BlockSpec((win, d), lambda i: (i, 0))],
            core_axis_name=("c", "s"),
            dimension_semantics=(pltpu.PARALLEL,),
        )(i_hbm, o_hbm)

    return k(table, idx2d)


def _sc_scatter(yx, di128):
    e = yx.shape[1]
    nch = e // 128
    assert nch % NS == 0
    trips = nch // NS
    stripe = NPAD // NS
    mesh = plsc.VectorSubcoreMesh(core_axis_name="c", subcore_axis_name="s")

    @functools.partial(
        pl.kernel, mesh=mesh,
        out_type=jax.ShapeDtypeStruct((NC, NPAD, 128), jnp.float32),
        scratch_types=[
            pltpu.VMEM((128, 128), jnp.float32),
            pltpu.VMEM((1, 128), jnp.int32),
            pltpu.VMEM_SHARED((NPAD, 128), jnp.float32),
        ])
    def k(yx_hbm, di_hbm, acc_out, ybuf, dibuf, acc_sh):
        c = lax.axis_index("c")
        s = lax.axis_index("s")
        zv = jnp.zeros((16,), jnp.float32)

        @pl.loop(0, 128)
        def _(r):
            @pl.loop(0, 128, step=16)
            def _(cc):
                ybuf[r, pl.ds(cc, 16)] = zv

        @pl.loop(0, stripe, step=128)
        def _(r0):
            pltpu.sync_copy(ybuf, acc_sh.at[pl.ds(s * stripe + r0, 128)])

        plsc.subcore_barrier()

        @pl.loop(0, trips)
        def _(t):
            ci = t * NS + s
            base = ci * 128
            pltpu.sync_copy(yx_hbm.at[c, pl.ds(base, 128)], ybuf)
            pltpu.sync_copy(di_hbm.at[pl.ds(ci, 1)], dibuf)
            pltpu.sync_copy(ybuf, acc_sh.at[dibuf.at[0]], add=True)

        plsc.subcore_barrier()

        @pl.loop(0, stripe, step=128)
        def _(r0):
            row = s * stripe + r0
            pltpu.sync_copy(acc_sh.at[pl.ds(row, 128)], ybuf)
            pltpu.sync_copy(ybuf, acc_out.at[c, pl.ds(row, 128)])

    return k(yx, di128)


def _fold(w, b, rel):
    wf = jnp.einsum("ihd,hde->ihe", w.reshape(HID, H, D), rel).reshape(HID, HID)
    bf = jnp.einsum("hd,hde->he", b.reshape(H, D), rel).reshape(HID)
    return wf, bf


def kernel(x_n1, x_n2, params, edge_index_n1_n2, edge_index_n2_n1, pair_index):
    p = params
    edge_types = (("n1", "n2", "n1__to__n2"), ("n2", "n1", "n2__to__n1"))
    edges = {"n1__to__n2": edge_index_n1_n2, "n2__to__n1": edge_index_n2_n1}

    xd = {"n1": _matmul(x_n1, p["W_in_n1"], p["b_in_n1"], act="relu"),
          "n2": _matmul(x_n2, p["W_in_n2"], p["b_in_n2"], act="relu")}

    outs = []
    for l in range(L):
        kv = {}
        q = {}
        for (src, dst, en) in edge_types:
            wkf, bkf = _fold(p[f"W_k_{l}_{src}"], p[f"b_k_{l}_{src}"],
                             p[f"arel_{l}_{en}"])
            wvf, bvf = _fold(p[f"W_v_{l}_{src}"], p[f"b_v_{l}_{src}"],
                             p[f"mrel_{l}_{en}"])
            kv[src] = _matmul(xd[src],
                              jnp.concatenate([wkf, wvf], axis=1),
                              jnp.concatenate([bkf, bvf]))
            q[dst] = _matmul(xd[dst], p[f"W_q_{l}_{dst}"], p[f"b_q_{l}_{dst}"])

        agg = {}
        for (src, dst, en) in edge_types:
            ei = edges[en].astype(jnp.int32)
            e = ei.shape[1]
            epad = -(-e // EB) * EB
            si2d = jnp.pad(ei[0], (0, epad - e)).reshape(1, epad)
            dig = jnp.pad(ei[1], (0, epad - e)).reshape(1, epad)
            dis = jnp.pad(ei[1], (0, epad - e), constant_values=NPAD - 1)
            kvg = _sc_gather(kv[src], si2d)
            qg = _sc_gather(q[dst], dig)
            prel16 = jnp.concatenate(
                [p[f"prel_{l}_{en}"] / jnp.sqrt(jnp.float32(D)),
                 jnp.zeros((8,), jnp.float32)]).reshape(1, 16)
            alpha16, gmax = _alpha(kvg, qg, prel16)
            yx = _weighted_v(alpha16, kvg, gmax)
            agg[dst] = _sc_scatter(yx, dis.reshape(epad // 128, 128))

        new_xd = {}
        for t in ("n1", "n2"):
            beta = jax.nn.sigmoid(p[f"skip_{l}_{t}"])
            new_xd[t] = _finalize(agg[t], xd[t],
                                  p[f"W_a_{l}_{t}"], p[f"b_a_{l}_{t}"], beta)
        xd = new_xd
        outs.append(dict(xd))

    t1 = jnp.concatenate([outs[0]["n1"], outs[1]["n1"]], axis=1)
    t2 = jnp.concatenate([outs[0]["n2"], outs[1]["n2"]], axis=1)
    npairs = pair_index.shape[1]
    npad = -(-npairs // 128) * 128
    pidx = pair_index.astype(jnp.int32)
    mi = jnp.pad(pidx[0], (0, npad - npairs)).reshape(1, npad)
    di = jnp.pad(pidx[1], (0, npad - npairs)).reshape(1, npad)
    g1 = _sc_gather(t1, mi)
    g2 = _sc_gather(t2, di)
    return _pair_dot(g1, g2, npairs)


# speedup vs baseline: 34.8237x; 1.0039x over previous
import functools

import jax
import jax.numpy as jnp
from jax import lax
from jax.experimental import pallas as pl
from jax.experimental.pallas import tpu as pltpu
from jax.experimental.pallas import tpu_sc as plsc

N = 10000
DF = 128
HID = 128
H = 8
D = 16
L = 2
NPAD = 10240
NC, NS = 2, 16
EB = 2048
NB = 1000


def _mm_body(x_ref, w_ref, b_ref, o_ref, *, act):
    y = jnp.dot(x_ref[...], w_ref[...], preferred_element_type=jnp.float32)
    y = y + b_ref[...]
    if act == "relu":
        y = jnp.maximum(y, 0.0)
    o_ref[...] = y


def _matmul(x, w, b, act=None, block=NB):
    n, fin = x.shape
    fout = w.shape[1]
    return pl.pallas_call(
        functools.partial(_mm_body, act=act),
        grid=(n // block,),
        in_specs=[pl.BlockSpec((block, fin), lambda i: (i, 0)),
                  pl.BlockSpec((fin, fout), lambda i: (0, 0)),
                  pl.BlockSpec((1, fout), lambda i: (0, 0))],
        out_specs=pl.BlockSpec((block, fout), lambda i: (i, 0)),
        out_shape=jax.ShapeDtypeStruct((n, fout), jnp.float32),
    )(x, w, b.reshape(1, fout))


def _head_sum_mat():
    r = lax.broadcasted_iota(jnp.int32, (HID, 16), 0) // D
    c = lax.broadcasted_iota(jnp.int32, (HID, 16), 1)
    return (r == c).astype(jnp.float32)


def _head_bcast_mat():
    r = lax.broadcasted_iota(jnp.int32, (16, HID), 0)
    c = lax.broadcasted_iota(jnp.int32, (16, HID), 1) // D
    return (r == c).astype(jnp.float32)


def _alpha_body(kg_ref, qg_ref, prel_ref, a_ref, m_ref):
    i = pl.program_id(0)
    prod = kg_ref[...] * qg_ref[...]
    a16 = jnp.dot(prod, _head_sum_mat(), preferred_element_type=jnp.float32)
    a16 = a16 * prel_ref[...]
    a_ref[...] = a16
    bm = jnp.max(a16).reshape(1, 1)

    @pl.when(i == 0)
    def _():
        m_ref[...] = bm

    @pl.when(i > 0)
    def _():
        m_ref[...] = jnp.maximum(m_ref[...], bm)


def _alpha(kvg, qg, prel16):
    e = qg.shape[0]
    return pl.pallas_call(
        _alpha_body,
        grid=(e // EB,),
        in_specs=[pl.BlockSpec((EB, HID), lambda i: (i, 0)),
                  pl.BlockSpec((EB, HID), lambda i: (i, 0)),
                  pl.BlockSpec((1, 16), lambda i: (0, 0))],
        out_specs=[pl.BlockSpec((EB, 16), lambda i: (i, 0)),
                   pl.BlockSpec((1, 1), lambda i: (0, 0))],
        out_shape=[jax.ShapeDtypeStruct((e, 16), jnp.float32),
                   jax.ShapeDtypeStruct((1, 1), jnp.float32)],
    )(kvg, qg, prel16)


def _wv_body(a_ref, vg_ref, m_ref, o_ref):
    ex16 = jnp.exp(a_ref[...] - m_ref[...])
    exfull = jnp.dot(ex16, _head_bcast_mat(), preferred_element_type=jnp.float32)
    o_ref[...] = jnp.stack([vg_ref[...] * exfull, exfull])


def _weighted_v(alpha16, kvg, gmax):
    e = alpha16.shape[0]
    return pl.pallas_call(
        _wv_body,
        grid=(e // EB,),
        in_specs=[pl.BlockSpec((EB, 16), lambda i: (i, 0)),
                  pl.BlockSpec((EB, HID), lambda i: (i, 1)),
                  pl.BlockSpec((1, 1), lambda i: (0, 0))],
        out_specs=pl.BlockSpec((2, EB, HID), lambda i: (0, i, 0)),
        out_shape=jax.ShapeDtypeStruct((2, e, HID), jnp.float32),
    )(alpha16, kvg, gmax)


def _fin_body(nd_ref, xd_ref, wa_ref, ba_ref, beta_ref, o_ref):
    parts = nd_ref[...]
    num = parts[0]
    denfull = parts[1]
    agg = num / (denfull + 1e-16)
    g = jax.nn.gelu(agg)
    a = jnp.dot(g, wa_ref[...], preferred_element_type=jnp.float32) + ba_ref[...]
    beta = beta_ref[...]
    o_ref[...] = beta * a + (1.0 - beta) * xd_ref[...]


def _finalize(numden, xd, wa, ba, beta):
    return pl.pallas_call(
        _fin_body,
        grid=(N // NB,),
        in_specs=[pl.BlockSpec((2, NB, HID), lambda i: (0, i, 0)),
                  pl.BlockSpec((NB, HID), lambda i: (i, 0)),
                  pl.BlockSpec((HID, HID), lambda i: (0, 0)),
                  pl.BlockSpec((1, HID), lambda i: (0, 0)),
                  pl.BlockSpec((1, 1), lambda i: (0, 0))],
        out_specs=pl.BlockSpec((NB, HID), lambda i: (i, 0)),
        out_shape=jax.ShapeDtypeStruct((N, HID), jnp.float32),
    )(numden, xd, wa, ba.reshape(1, HID), beta.reshape(1, 1))


def _dot_body(g1_ref, g2_ref, o_ref):
    o_ref[...] = jnp.sum(g1_ref[...] * g2_ref[...], axis=1, keepdims=True)


def _pair_dot(g1, g2, npairs, block=NB):
    width = g1.shape[1]
    return pl.pallas_call(
        _dot_body,
        grid=(npairs // block,),
        in_specs=[pl.BlockSpec((block, width), lambda i: (i, 0)),
                  pl.BlockSpec((block, width), lambda i: (i, 0))],
        out_specs=pl.BlockSpec((block, 1), lambda i: (i, 0)),
        out_shape=jax.ShapeDtypeStruct((npairs, 1), jnp.float32),
    )(g1, g2)


def _sc_gather(table, idx128):
    n, d = table.shape
    nch = idx128.shape[0]
    e = nch * 128
    wpr = 2 if d <= 128 else 1
    win = 128 * wpr
    mesh = plsc.VectorSubcoreMesh(core_axis_name="c", subcore_axis_name="s")

    @functools.partial(pl.kernel, mesh=mesh,
                       out_type=jax.ShapeDtypeStruct((e, d), table.dtype))
    def k(tab_hbm, i_hbm, o_hbm):
        def body(i_vmem, o_vmem):
            for j in range(wpr):
                pltpu.sync_copy(tab_hbm.at[i_vmem.at[j]],
                                o_vmem.at[pl.ds(j * 128, 128)])

        pltpu.emit_pipeline(
            body,
            grid=(e // win,),
            in_specs=[pl.BlockSpec((wpr, 128), lambda i: (i, 0))],
            out_specs=[pl.BlockSpec((win, d), lambda i: (i, 0))],
            core_axis_name=("c", "s"),
            dimension_semantics=(pltpu.PARALLEL,),
        )(i_hbm, o_hbm)

    return k(table, idx128)


def _sc_scatter(yx, di128):
    e = yx.shape[1]
    nch = e // 128
    assert nch % NS == 0
    trips = nch // NS
    stripe = NPAD // NS
    mesh = plsc.VectorSubcoreMesh(core_axis_name="c", subcore_axis_name="s")

    @functools.partial(
        pl.kernel, mesh=mesh,
        out_type=jax.ShapeDtypeStruct((NC, NPAD, 128), jnp.float32),
        scratch_types=[
            pltpu.VMEM((128, 128), jnp.float32),
            pltpu.VMEM((1, 128), jnp.int32),
            pltpu.VMEM_SHARED((NPAD, 128), jnp.float32),
        ])
    def k(yx_hbm, di_hbm, acc_out, ybuf, dibuf, acc_sh):
        c = lax.axis_index("c")
        s = lax.axis_index("s")
        zv = jnp.zeros((16,), jnp.float32)

        @pl.loop(0, 128)
        def _(r):
            @pl.loop(0, 128, step=16)
            def _(cc):
                ybuf[r, pl.ds(cc, 16)] = zv

        @pl.loop(0, stripe, step=128)
        def _(r0):
            pltpu.sync_copy(ybuf, acc_sh.at[pl.ds(s * stripe + r0, 128)])

        plsc.subcore_barrier()

        @pl.loop(0, trips)
        def _(t):
            ci = t * NS + s
            base = ci * 128
            pltpu.sync_copy(yx_hbm.at[c, pl.ds(base, 128)], ybuf)
            pltpu.sync_copy(di_hbm.at[pl.ds(ci, 1)], dibuf)
            pltpu.sync_copy(ybuf, acc_sh.at[dibuf.at[0]], add=True)

        plsc.subcore_barrier()

        @pl.loop(0, stripe, step=128)
        def _(r0):
            row = s * stripe + r0
            pltpu.sync_copy(acc_sh.at[pl.ds(row, 128)], ybuf)
            pltpu.sync_copy(ybuf, acc_out.at[c, pl.ds(row, 128)])

    return k(yx, di128)


def _fold(w, b, rel):
    wf = jnp.einsum("ihd,hde->ihe", w.reshape(HID, H, D), rel).reshape(HID, HID)
    bf = jnp.einsum("hd,hde->he", b.reshape(H, D), rel).reshape(HID)
    return wf, bf


def kernel(x_n1, x_n2, params, edge_index_n1_n2, edge_index_n2_n1, pair_index):
    p = params
    edge_types = (("n1", "n2", "n1__to__n2"), ("n2", "n1", "n2__to__n1"))
    edges = {"n1__to__n2": edge_index_n1_n2, "n2__to__n1": edge_index_n2_n1}

    xd = {"n1": _matmul(x_n1, p["W_in_n1"], p["b_in_n1"], act="relu"),
          "n2": _matmul(x_n2, p["W_in_n2"], p["b_in_n2"], act="relu")}

    outs = []
    for l in range(L):
        kv = {}
        q = {}
        for (src, dst, en) in edge_types:
            wkf, bkf = _fold(p[f"W_k_{l}_{src}"], p[f"b_k_{l}_{src}"],
                             p[f"arel_{l}_{en}"])
            wvf, bvf = _fold(p[f"W_v_{l}_{src}"], p[f"b_v_{l}_{src}"],
                             p[f"mrel_{l}_{en}"])
            kv[src] = _matmul(xd[src],
                              jnp.concatenate([wkf, wvf], axis=1),
                              jnp.concatenate([bkf, bvf]))
            q[dst] = _matmul(xd[dst], p[f"W_q_{l}_{dst}"], p[f"b_q_{l}_{dst}"])

        agg = {}
        for (src, dst, en) in edge_types:
            ei = edges[en].astype(jnp.int32)
            e = ei.shape[1]
            epad = -(-e // EB) * EB
            si2d = jnp.pad(ei[0], (0, epad - e)).reshape(epad // 128, 128)
            dig = jnp.pad(ei[1], (0, epad - e)).reshape(epad // 128, 128)
            dis = jnp.pad(ei[1], (0, epad - e), constant_values=NPAD - 1)
            kvg = _sc_gather(kv[src], si2d)
            qg = _sc_gather(q[dst], dig)
            prel16 = jnp.concatenate(
                [p[f"prel_{l}_{en}"] / jnp.sqrt(jnp.float32(D)),
                 jnp.zeros((8,), jnp.float32)]).reshape(1, 16)
            alpha16, gmax = _alpha(kvg, qg, prel16)
            yx = _weighted_v(alpha16, kvg, gmax)
            agg[dst] = _sc_scatter(yx, dis.reshape(epad // 128, 128))

        new_xd = {}
        for t in ("n1", "n2"):
            beta = jax.nn.sigmoid(p[f"skip_{l}_{t}"])
            new_xd[t] = _finalize(agg[t], xd[t],
                                  p[f"W_a_{l}_{t}"], p[f"b_a_{l}_{t}"], beta)
        xd = new_xd
        outs.append(dict(xd))

    t1 = jnp.concatenate([outs[0]["n1"], outs[1]["n1"]], axis=1)
    t2 = jnp.concatenate([outs[0]["n2"], outs[1]["n2"]], axis=1)
    npairs = pair_index.shape[1]
    npad = -(-npairs // 128) * 128
    pidx = pair_index.astype(jnp.int32)
    mi = jnp.pad(pidx[0], (0, npad - npairs)).reshape(npad // 128, 128)
    di = jnp.pad(pidx[1], (0, npad - npairs)).reshape(npad // 128, 128)
    g1 = _sc_gather(t1, mi)
    g2 = _sc_gather(t2, di)
    return _pair_dot(g1, g2, npairs)


# speedup vs baseline: 43.4132x; 1.2467x over previous
import functools

import jax
import jax.numpy as jnp
from jax import lax
from jax.experimental import pallas as pl
from jax.experimental.pallas import tpu as pltpu
from jax.experimental.pallas import tpu_sc as plsc

N = 10000
DF = 128
HID = 128
H = 8
D = 16
L = 2
NPAD = 10240
NC, NS = 2, 16
EB = 2048
NB = 1000


def _mm_body(x_ref, w_ref, b_ref, o_ref, *, act):
    y = jnp.dot(x_ref[...], w_ref[...], preferred_element_type=jnp.float32)
    y = y + b_ref[...]
    if act == "relu":
        y = jnp.maximum(y, 0.0)
    o_ref[...] = y


def _matmul(x, w, b, act=None, block=NB):
    n, fin = x.shape
    fout = w.shape[1]
    return pl.pallas_call(
        functools.partial(_mm_body, act=act),
        grid=(n // block,),
        in_specs=[pl.BlockSpec((block, fin), lambda i: (i, 0)),
                  pl.BlockSpec((fin, fout), lambda i: (0, 0)),
                  pl.BlockSpec((1, fout), lambda i: (0, 0))],
        out_specs=pl.BlockSpec((block, fout), lambda i: (i, 0)),
        out_shape=jax.ShapeDtypeStruct((n, fout), jnp.float32),
    )(x, w, b.reshape(1, fout))


def _head_sum_mat():
    r = lax.broadcasted_iota(jnp.int32, (HID, 16), 0) // D
    c = lax.broadcasted_iota(jnp.int32, (HID, 16), 1)
    return (r == c).astype(jnp.float32)


def _head_bcast_mat():
    r = lax.broadcasted_iota(jnp.int32, (16, HID), 0)
    c = lax.broadcasted_iota(jnp.int32, (16, HID), 1) // D
    return (r == c).astype(jnp.float32)


def _alpha_body(kg_ref, qg_ref, prel_ref, a_ref, m_ref):
    i = pl.program_id(0)
    prod = kg_ref[...] * qg_ref[...]
    a16 = jnp.dot(prod, _head_sum_mat(), preferred_element_type=jnp.float32)
    a16 = a16 * prel_ref[...]
    a_ref[...] = a16
    bm = jnp.max(a16).reshape(1, 1)

    @pl.when(i == 0)
    def _():
        m_ref[...] = bm

    @pl.when(i > 0)
    def _():
        m_ref[...] = jnp.maximum(m_ref[...], bm)


def _alpha(kvg, qg, prel16):
    e = qg.shape[0]
    return pl.pallas_call(
        _alpha_body,
        grid=(e // EB,),
        in_specs=[pl.BlockSpec((EB, HID), lambda i: (i, 0)),
                  pl.BlockSpec((EB, HID), lambda i: (i, 0)),
                  pl.BlockSpec((1, 16), lambda i: (0, 0))],
        out_specs=[pl.BlockSpec((EB, 16), lambda i: (i, 0)),
                   pl.BlockSpec((1, 1), lambda i: (0, 0))],
        out_shape=[jax.ShapeDtypeStruct((e, 16), jnp.float32),
                   jax.ShapeDtypeStruct((1, 1), jnp.float32)],
    )(kvg, qg, prel16)


def _wv_body(a_ref, vg_ref, m_ref, o_ref):
    ex16 = jnp.exp(a_ref[...] - m_ref[...])
    exfull = jnp.dot(ex16, _head_bcast_mat(), preferred_element_type=jnp.float32)
    o_ref[...] = jnp.stack([vg_ref[...] * exfull, exfull])


def _weighted_v(alpha16, kvg, gmax):
    e = alpha16.shape[0]
    return pl.pallas_call(
        _wv_body,
        grid=(e // EB,),
        in_specs=[pl.BlockSpec((EB, 16), lambda i: (i, 0)),
                  pl.BlockSpec((EB, HID), lambda i: (i, 1)),
                  pl.BlockSpec((1, 1), lambda i: (0, 0))],
        out_specs=pl.BlockSpec((2, EB, HID), lambda i: (0, i, 0)),
        out_shape=jax.ShapeDtypeStruct((2, e, HID), jnp.float32),
    )(alpha16, kvg, gmax)


def _fin_body(nd_ref, xd_ref, wa_ref, ba_ref, beta_ref, o_ref):
    parts = nd_ref[...]
    num = parts[0]
    denfull = parts[1]
    agg = num / (denfull + 1e-16)
    g = jax.nn.gelu(agg)
    a = jnp.dot(g, wa_ref[...], preferred_element_type=jnp.float32) + ba_ref[...]
    beta = beta_ref[...]
    o_ref[...] = beta * a + (1.0 - beta) * xd_ref[...]


def _finalize(numden, xd, wa, ba, beta):
    return pl.pallas_call(
        _fin_body,
        grid=(N // NB,),
        in_specs=[pl.BlockSpec((2, NB, HID), lambda i: (0, i, 0)),
                  pl.BlockSpec((NB, HID), lambda i: (i, 0)),
                  pl.BlockSpec((HID, HID), lambda i: (0, 0)),
                  pl.BlockSpec((1, HID), lambda i: (0, 0)),
                  pl.BlockSpec((1, 1), lambda i: (0, 0))],
        out_specs=pl.BlockSpec((NB, HID), lambda i: (i, 0)),
        out_shape=jax.ShapeDtypeStruct((N, HID), jnp.float32),
    )(numden, xd, wa, ba.reshape(1, HID), beta.reshape(1, 1))


def _dot_body(g1_ref, g2_ref, o_ref):
    o_ref[...] = jnp.sum(g1_ref[...] * g2_ref[...], axis=1, keepdims=True)


def _pair_dot(g1, g2, npairs, block=NB):
    width = g1.shape[1]
    return pl.pallas_call(
        _dot_body,
        grid=(npairs // block,),
        in_specs=[pl.BlockSpec((block, width), lambda i: (i, 0)),
                  pl.BlockSpec((block, width), lambda i: (i, 0))],
        out_specs=pl.BlockSpec((block, 1), lambda i: (i, 0)),
        out_shape=jax.ShapeDtypeStruct((npairs, 1), jnp.float32),
    )(g1, g2)


def _sc_gather(table, idx128):
    n, d = table.shape
    nch = idx128.shape[0]
    e = nch * 128
    wpr = 1
    win = 128 * wpr
    mesh = plsc.VectorSubcoreMesh(core_axis_name="c", subcore_axis_name="s")

    @functools.partial(pl.kernel, mesh=mesh,
                       out_type=jax.ShapeDtypeStruct((e, d), table.dtype))
    def k(tab_hbm, i_hbm, o_hbm):
        def body(i_vmem, o_vmem):
            for j in range(wpr):
                pltpu.sync_copy(tab_hbm.at[i_vmem.at[j]],
                                o_vmem.at[pl.ds(j * 128, 128)])

        pltpu.emit_pipeline(
            body,
            grid=(e // win,),
            in_specs=[pl.BlockSpec((wpr, 128), lambda i: (i, 0))],
            out_specs=[pl.BlockSpec((win, d), lambda i: (i, 0))],
            core_axis_name=("c", "s"),
            dimension_semantics=(pltpu.PARALLEL,),
        )(i_hbm, o_hbm)

    return k(table, idx128)


def _sc_scatter(yx, di128):
    e = yx.shape[1]
    nch = e // 128
    assert nch % NS == 0
    trips = nch // NS
    stripe = NPAD // NS
    mesh = plsc.VectorSubcoreMesh(core_axis_name="c", subcore_axis_name="s")

    @functools.partial(
        pl.kernel, mesh=mesh,
        out_type=jax.ShapeDtypeStruct((NC, NPAD, 128), jnp.float32),
        scratch_types=[
            pltpu.VMEM((128, 128), jnp.float32),
            pltpu.VMEM((1, 128), jnp.int32),
            pltpu.VMEM_SHARED((NPAD, 128), jnp.float32),
        ])
    def k(yx_hbm, di_hbm, acc_out, ybuf, dibuf, acc_sh):
        c = lax.axis_index("c")
        s = lax.axis_index("s")
        zv = jnp.zeros((16,), jnp.float32)

        @pl.loop(0, 128)
        def _(r):
            @pl.loop(0, 128, step=16)
            def _(cc):
                ybuf[r, pl.ds(cc, 16)] = zv

        @pl.loop(0, stripe, step=128)
        def _(r0):
            pltpu.sync_copy(ybuf, acc_sh.at[pl.ds(s * stripe + r0, 128)])

        plsc.subcore_barrier()

        @pl.loop(0, trips)
        def _(t):
            ci = t * NS + s
            base = ci * 128
            pltpu.sync_copy(yx_hbm.at[c, pl.ds(base, 128)], ybuf)
            pltpu.sync_copy(di_hbm.at[pl.ds(ci, 1)], dibuf)
            pltpu.sync_copy(ybuf, acc_sh.at[dibuf.at[0]], add=True)

        plsc.subcore_barrier()

        @pl.loop(0, stripe, step=128)
        def _(r0):
            row = s * stripe + r0
            pltpu.sync_copy(acc_sh.at[pl.ds(row, 128)], ybuf)
            pltpu.sync_copy(ybuf, acc_out.at[c, pl.ds(row, 128)])

    return k(yx, di128)


def _fold(w, b, rel):
    wf = jnp.einsum("ihd,hde->ihe", w.reshape(HID, H, D), rel).reshape(HID, HID)
    bf = jnp.einsum("hd,hde->he", b.reshape(H, D), rel).reshape(HID)
    return wf, bf


def kernel(x_n1, x_n2, params, edge_index_n1_n2, edge_index_n2_n1, pair_index):
    p = params
    edge_types = (("n1", "n2", "n1__to__n2"), ("n2", "n1", "n2__to__n1"))
    edges = {"n1__to__n2": edge_index_n1_n2, "n2__to__n1": edge_index_n2_n1}

    xd = {"n1": _matmul(x_n1, p["W_in_n1"], p["b_in_n1"], act="relu"),
          "n2": _matmul(x_n2, p["W_in_n2"], p["b_in_n2"], act="relu")}

    outs = []
    for l in range(L):
        kv = {}
        q = {}
        for (src, dst, en) in edge_types:
            wkf, bkf = _fold(p[f"W_k_{l}_{src}"], p[f"b_k_{l}_{src}"],
                             p[f"arel_{l}_{en}"])
            wvf, bvf = _fold(p[f"W_v_{l}_{src}"], p[f"b_v_{l}_{src}"],
                             p[f"mrel_{l}_{en}"])
            kv[src] = _matmul(xd[src],
                              jnp.concatenate([wkf, wvf], axis=1),
                              jnp.concatenate([bkf, bvf]))
            q[dst] = _matmul(xd[dst], p[f"W_q_{l}_{dst}"], p[f"b_q_{l}_{dst}"])

        agg = {}
        for (src, dst, en) in edge_types:
            ei = edges[en].astype(jnp.int32)
            e = ei.shape[1]
            epad = -(-e // EB) * EB
            si2d = jnp.pad(ei[0], (0, epad - e)).reshape(epad // 128, 128)
            dig = jnp.pad(ei[1], (0, epad - e)).reshape(epad // 128, 128)
            dis = jnp.pad(ei[1], (0, epad - e), constant_values=NPAD - 1)
            kvg = _sc_gather(kv[src], si2d)
            qg = _sc_gather(q[dst], dig)
            prel16 = jnp.concatenate(
                [p[f"prel_{l}_{en}"] / jnp.sqrt(jnp.float32(D)),
                 jnp.zeros((8,), jnp.float32)]).reshape(1, 16)
            alpha16, gmax = _alpha(kvg, qg, prel16)
            yx = _weighted_v(alpha16, kvg, gmax)
            agg[dst] = _sc_scatter(yx, dis.reshape(epad // 128, 128))

        new_xd = {}
        for t in ("n1", "n2"):
            beta = jax.nn.sigmoid(p[f"skip_{l}_{t}"])
            new_xd[t] = _finalize(agg[t], xd[t],
                                  p[f"W_a_{l}_{t}"], p[f"b_a_{l}_{t}"], beta)
        xd = new_xd
        outs.append(dict(xd))

    t1 = jnp.concatenate([outs[0]["n1"], outs[1]["n1"]], axis=1)
    t2 = jnp.concatenate([outs[0]["n2"], outs[1]["n2"]], axis=1)
    npairs = pair_index.shape[1]
    npad = -(-npairs // 128) * 128
    pidx = pair_index.astype(jnp.int32)
    mi = jnp.pad(pidx[0], (0, npad - npairs)).reshape(npad // 128, 128)
    di = jnp.pad(pidx[1], (0, npad - npairs)).reshape(npad // 128, 128)
    g1 = _sc_gather(t1, mi)
    g2 = _sc_gather(t2, di)
    return _pair_dot(g1, g2, npairs)


# speedup vs baseline: 47.8782x; 1.1029x over previous
import functools

import jax
import jax.numpy as jnp
from jax import lax
from jax.experimental import pallas as pl
from jax.experimental.pallas import tpu as pltpu
from jax.experimental.pallas import tpu_sc as plsc

N = 10000
DF = 128
HID = 128
H = 8
D = 16
L = 2
NPAD = 10240
NC, NS = 2, 16
EB = 2048
NB = 1000


def _mm_body(x_ref, w_ref, b_ref, o_ref, *, act):
    y = jnp.dot(x_ref[...], w_ref[...], preferred_element_type=jnp.float32)
    y = y + b_ref[...]
    if act == "relu":
        y = jnp.maximum(y, 0.0)
    o_ref[...] = y


def _matmul(x, w, b, act=None, block=NB):
    n, fin = x.shape
    fout = w.shape[1]
    return pl.pallas_call(
        functools.partial(_mm_body, act=act),
        grid=(n // block,),
        in_specs=[pl.BlockSpec((block, fin), lambda i: (i, 0)),
                  pl.BlockSpec((fin, fout), lambda i: (0, 0)),
                  pl.BlockSpec((1, fout), lambda i: (0, 0))],
        out_specs=pl.BlockSpec((block, fout), lambda i: (i, 0)),
        out_shape=jax.ShapeDtypeStruct((n, fout), jnp.float32),
    )(x, w, b.reshape(1, fout))


def _lo_f32(xi):
    return jax.lax.bitcast_convert_type(xi << jnp.uint32(16), jnp.float32)


def _hi_f32(xi):
    return jax.lax.bitcast_convert_type(xi & jnp.uint32(0xFFFF0000), jnp.float32)


def _head_sum_mat():
    r = lax.broadcasted_iota(jnp.int32, (HID, 16), 0) // D
    c = lax.broadcasted_iota(jnp.int32, (HID, 16), 1)
    return (r == c).astype(jnp.float32)


def _head_bcast_mat():
    r = lax.broadcasted_iota(jnp.int32, (16, HID), 0)
    c = lax.broadcasted_iota(jnp.int32, (16, HID), 1) // D
    return (r == c).astype(jnp.float32)


def _alpha_body(kg_ref, qg_ref, prel_ref, a_ref, m_ref):
    i = pl.program_id(0)
    kg = _lo_f32(kg_ref[...])
    prod = kg * qg_ref[...]
    a16 = jnp.dot(prod, _head_sum_mat(), preferred_element_type=jnp.float32)
    a16 = a16 * prel_ref[...]
    a_ref[...] = a16
    bm = jnp.max(a16).reshape(1, 1)

    @pl.when(i == 0)
    def _():
        m_ref[...] = bm

    @pl.when(i > 0)
    def _():
        m_ref[...] = jnp.maximum(m_ref[...], bm)


def _alpha(kvg, qg, prel16):
    e = qg.shape[0]
    return pl.pallas_call(
        _alpha_body,
        grid=(e // EB,),
        in_specs=[pl.BlockSpec((EB, HID), lambda i: (i, 0)),
                  pl.BlockSpec((EB, HID), lambda i: (i, 0)),
                  pl.BlockSpec((1, 16), lambda i: (0, 0))],
        out_specs=[pl.BlockSpec((EB, 16), lambda i: (i, 0)),
                   pl.BlockSpec((1, 1), lambda i: (0, 0))],
        out_shape=[jax.ShapeDtypeStruct((e, 16), jnp.float32),
                   jax.ShapeDtypeStruct((1, 1), jnp.float32)],
    )(kvg, qg, prel16)


def _wv_body(a_ref, vg_ref, m_ref, o_ref):
    ex16 = jnp.exp(a_ref[...] - m_ref[...])
    exfull = jnp.dot(ex16, _head_bcast_mat(), preferred_element_type=jnp.float32)
    vg = _hi_f32(vg_ref[...])
    o_ref[...] = jnp.stack([vg * exfull, exfull])


def _weighted_v(alpha16, kvg, gmax):
    e = alpha16.shape[0]
    return pl.pallas_call(
        _wv_body,
        grid=(e // EB,),
        in_specs=[pl.BlockSpec((EB, 16), lambda i: (i, 0)),
                  pl.BlockSpec((EB, HID), lambda i: (i, 0)),
                  pl.BlockSpec((1, 1), lambda i: (0, 0))],
        out_specs=pl.BlockSpec((2, EB, HID), lambda i: (0, i, 0)),
        out_shape=jax.ShapeDtypeStruct((2, e, HID), jnp.float32),
    )(alpha16, kvg, gmax)


def _fin_body(nd_ref, xd_ref, wa_ref, ba_ref, beta_ref, o_ref):
    parts = nd_ref[...]
    num = parts[0]
    denfull = parts[1]
    agg = num / (denfull + 1e-16)
    g = jax.nn.gelu(agg)
    a = jnp.dot(g, wa_ref[...], preferred_element_type=jnp.float32) + ba_ref[...]
    beta = beta_ref[...]
    o_ref[...] = beta * a + (1.0 - beta) * xd_ref[...]


def _finalize(numden, xd, wa, ba, beta):
    return pl.pallas_call(
        _fin_body,
        grid=(N // NB,),
        in_specs=[pl.BlockSpec((2, NB, HID), lambda i: (0, i, 0)),
                  pl.BlockSpec((NB, HID), lambda i: (i, 0)),
                  pl.BlockSpec((HID, HID), lambda i: (0, 0)),
                  pl.BlockSpec((1, HID), lambda i: (0, 0)),
                  pl.BlockSpec((1, 1), lambda i: (0, 0))],
        out_specs=pl.BlockSpec((NB, HID), lambda i: (i, 0)),
        out_shape=jax.ShapeDtypeStruct((N, HID), jnp.float32),
    )(numden, xd, wa, ba.reshape(1, HID), beta.reshape(1, 1))


def _dot_body(g1_ref, g2_ref, o_ref):
    g1 = g1_ref[...]
    g2 = g2_ref[...]
    prod = _lo_f32(g1) * _lo_f32(g2) + _hi_f32(g1) * _hi_f32(g2)
    o_ref[...] = jnp.sum(prod, axis=1, keepdims=True)


def _pair_dot(g1, g2, npairs, block=NB):
    width = g1.shape[1]
    return pl.pallas_call(
        _dot_body,
        grid=(npairs // block,),
        in_specs=[pl.BlockSpec((block, width), lambda i: (i, 0)),
                  pl.BlockSpec((block, width), lambda i: (i, 0))],
        out_specs=pl.BlockSpec((block, 1), lambda i: (i, 0)),
        out_shape=jax.ShapeDtypeStruct((npairs, 1), jnp.float32),
    )(g1, g2)


def _sc_gather(table, idx128):
    n, d = table.shape
    nch = idx128.shape[0]
    e = nch * 128
    wpr = 1
    win = 128 * wpr
    mesh = plsc.VectorSubcoreMesh(core_axis_name="c", subcore_axis_name="s")

    @functools.partial(pl.kernel, mesh=mesh,
                       out_type=jax.ShapeDtypeStruct((e, d), table.dtype))
    def k(tab_hbm, i_hbm, o_hbm):
        def body(i_vmem, o_vmem):
            for j in range(wpr):
                pltpu.sync_copy(tab_hbm.at[i_vmem.at[j]],
                                o_vmem.at[pl.ds(j * 128, 128)])

        pltpu.emit_pipeline(
            body,
            grid=(e // win,),
            in_specs=[pl.BlockSpec((wpr, 128), lambda i: (i, 0))],
            out_specs=[pl.BlockSpec((win, d), lambda i: (i, 0))],
            core_axis_name=("c", "s"),
            dimension_semantics=(pltpu.PARALLEL,),
        )(i_hbm, o_hbm)

    return k(table, idx128)


def _sc_scatter(yx, di128):
    e = yx.shape[1]
    nch = e // 128
    assert nch % NS == 0
    trips = nch // NS
    stripe = NPAD // NS
    mesh = plsc.VectorSubcoreMesh(core_axis_name="c", subcore_axis_name="s")

    @functools.partial(
        pl.kernel, mesh=mesh,
        out_type=jax.ShapeDtypeStruct((NC, NPAD, 128), jnp.float32),
        scratch_types=[
            pltpu.VMEM((128, 128), jnp.float32),
            pltpu.VMEM((1, 128), jnp.int32),
            pltpu.VMEM_SHARED((NPAD, 128), jnp.float32),
        ])
    def k(yx_hbm, di_hbm, acc_out, ybuf, dibuf, acc_sh):
        c = lax.axis_index("c")
        s = lax.axis_index("s")
        zv = jnp.zeros((16,), jnp.float32)

        @pl.loop(0, 128)
        def _(r):
            @pl.loop(0, 128, step=16)
            def _(cc):
                ybuf[r, pl.ds(cc, 16)] = zv

        @pl.loop(0, stripe, step=128)
        def _(r0):
            pltpu.sync_copy(ybuf, acc_sh.at[pl.ds(s * stripe + r0, 128)])

        plsc.subcore_barrier()

        @pl.loop(0, trips)
        def _(t):
            ci = t * NS + s
            base = ci * 128
            pltpu.sync_copy(yx_hbm.at[c, pl.ds(base, 128)], ybuf)
            pltpu.sync_copy(di_hbm.at[pl.ds(ci, 1)], dibuf)
            pltpu.sync_copy(ybuf, acc_sh.at[dibuf.at[0]], add=True)

        plsc.subcore_barrier()

        @pl.loop(0, stripe, step=128)
        def _(r0):
            row = s * stripe + r0
            pltpu.sync_copy(acc_sh.at[pl.ds(row, 128)], ybuf)
            pltpu.sync_copy(ybuf, acc_out.at[c, pl.ds(row, 128)])

    return k(yx, di128)


def _pack_bf16(x):
    n, w = x.shape
    k = w // 2
    bits = jax.lax.bitcast_convert_type(x.astype(jnp.bfloat16), jnp.uint16)
    lo = bits[:, :k].astype(jnp.uint32)
    hi = bits[:, k:].astype(jnp.uint32)
    return lo | (hi << jnp.uint32(16))


def _fold(w, b, rel):
    wf = jnp.einsum("ihd,hde->ihe", w.reshape(HID, H, D), rel).reshape(HID, HID)
    bf = jnp.einsum("hd,hde->he", b.reshape(H, D), rel).reshape(HID)
    return wf, bf


def kernel(x_n1, x_n2, params, edge_index_n1_n2, edge_index_n2_n1, pair_index):
    p = params
    edge_types = (("n1", "n2", "n1__to__n2"), ("n2", "n1", "n2__to__n1"))
    edges = {"n1__to__n2": edge_index_n1_n2, "n2__to__n1": edge_index_n2_n1}

    xd = {"n1": _matmul(x_n1, p["W_in_n1"], p["b_in_n1"], act="relu"),
          "n2": _matmul(x_n2, p["W_in_n2"], p["b_in_n2"], act="relu")}

    outs = []
    for l in range(L):
        kv = {}
        q = {}
        for (src, dst, en) in edge_types:
            wkf, bkf = _fold(p[f"W_k_{l}_{src}"], p[f"b_k_{l}_{src}"],
                             p[f"arel_{l}_{en}"])
            wvf, bvf = _fold(p[f"W_v_{l}_{src}"], p[f"b_v_{l}_{src}"],
                             p[f"mrel_{l}_{en}"])
            kv[src] = _pack_bf16(_matmul(xd[src],
                                         jnp.concatenate([wkf, wvf], axis=1),
                                         jnp.concatenate([bkf, bvf])))
            q[dst] = _matmul(xd[dst], p[f"W_q_{l}_{dst}"], p[f"b_q_{l}_{dst}"])

        agg = {}
        for (src, dst, en) in edge_types:
            ei = edges[en].astype(jnp.int32)
            e = ei.shape[1]
            epad = -(-e // EB) * EB
            si2d = jnp.pad(ei[0], (0, epad - e)).reshape(epad // 128, 128)
            dig = jnp.pad(ei[1], (0, epad - e)).reshape(epad // 128, 128)
            dis = jnp.pad(ei[1], (0, epad - e), constant_values=NPAD - 1)
            kvg = _sc_gather(kv[src], si2d)
            qg = _sc_gather(q[dst], dig)
            prel16 = jnp.concatenate(
                [p[f"prel_{l}_{en}"] / jnp.sqrt(jnp.float32(D)),
                 jnp.zeros((8,), jnp.float32)]).reshape(1, 16)
            alpha16, gmax = _alpha(kvg, qg, prel16)
            yx = _weighted_v(alpha16, kvg, gmax)
            agg[dst] = _sc_scatter(yx, dis.reshape(epad // 128, 128))

        new_xd = {}
        for t in ("n1", "n2"):
            beta = jax.nn.sigmoid(p[f"skip_{l}_{t}"])
            new_xd[t] = _finalize(agg[t], xd[t],
                                  p[f"W_a_{l}_{t}"], p[f"b_a_{l}_{t}"], beta)
        xd = new_xd
        outs.append(dict(xd))

    t1 = _pack_bf16(jnp.concatenate([outs[0]["n1"], outs[1]["n1"]], axis=1))
    t2 = _pack_bf16(jnp.concatenate([outs[0]["n2"], outs[1]["n2"]], axis=1))
    npairs = pair_index.shape[1]
    npad = -(-npairs // 128) * 128
    pidx = pair_index.astype(jnp.int32)
    mi = jnp.pad(pidx[0], (0, npad - npairs)).reshape(npad // 128, 128)
    di = jnp.pad(pidx[1], (0, npad - npairs)).reshape(npad // 128, 128)
    g1 = _sc_gather(t1, mi)
    g2 = _sc_gather(t2, di)
    return _pair_dot(g1, g2, npairs)
